# Initial kernel scaffold; baseline (speedup 1.0000x reference)
import jax
import jax.numpy as jnp
from jax.experimental import pallas as pl


def kernel(x, table, W, b):
    raise NotImplementedError("write your pallas kernel here")


# speedup vs baseline: 1.4385x; 1.4385x over previous
import functools

import jax
import jax.numpy as jnp
from jax import lax
from jax.experimental import pallas as pl
from jax.experimental.pallas import tpu as pltpu
from jax.experimental.pallas import tpu_sc as plsc

_VOCAB = 100000
_HID = 64
_BATCH = 1024
_SEQ = 200

_NC = 2
_NS = 16
_L = 16
_NW = _NC * _NS
_RPW = _BATCH // _NW
_NCOL = _HID // _L

_sc_mesh = plsc.VectorSubcoreMesh(
    core_axis_name="c", subcore_axis_name="s", num_cores=_NC, num_subcores=_NS
)


@functools.partial(
    pl.kernel,
    out_type=jax.ShapeDtypeStruct((_BATCH, _HID), jnp.float32),
    mesh=_sc_mesh,
    scratch_types=[
        pltpu.VMEM((_RPW * _SEQ,), jnp.int32),
        pltpu.VMEM((2, _SEQ, _HID), jnp.float32),
        pltpu.VMEM((_RPW, _HID), jnp.float32),
        pltpu.SemaphoreType.DMA,
        pltpu.SemaphoreType.DMA,
    ],
    compiler_params=pltpu.CompilerParams(use_tc_tiling_on_sc=False),
)
def _sc_pooled_sum(idx_hbm, table_hbm, out_hbm, idx_v, rows_v, pool_v, sem0, sem1):
    wid = lax.axis_index("s") * _NC + lax.axis_index("c")
    row_base = wid * _RPW
    pltpu.sync_copy(idx_hbm.at[pl.ds(row_base * _SEQ, _RPW * _SEQ)], idx_v)

    sems = (sem0, sem1)

    def fire(g):
        buf = g % 2
        return pltpu.async_copy(
            table_hbm.at[idx_v.at[pl.ds(g * _SEQ, _SEQ)]],
            rows_v.at[buf],
            sems[buf],
        )

    def reduce_into(g):
        buf = g % 2

        def body(j8, acc):
            for k in range(8):
                j = j8 * 8 + k
                acc = tuple(
                    acc[c] + rows_v[buf, j, pl.ds(c * _L, _L)]
                    for c in range(_NCOL)
                )
            return acc

        z = jnp.zeros((_L,), jnp.float32)
        acc = lax.fori_loop(0, _SEQ // 8, body, (z,) * _NCOL)
        for c in range(_NCOL):
            pool_v[g, pl.ds(c * _L, _L)] = acc[c]

    cps = [None, None]
    cps[0] = fire(0)
    for g in range(_RPW):
        if g + 1 < _RPW:
            cps[(g + 1) % 2] = fire(g + 1)
        cps[g % 2].wait()
        reduce_into(g)

    pltpu.sync_copy(pool_v, out_hbm.at[pl.ds(row_base, _RPW)])


_VBLK = 2048


def _linear_body(vec_ref, w_ref, b_ref, out_ref):
    v = jnp.tanh(vec_ref[...] * (1.0 / _SEQ))
    out_ref[...] = (
        lax.dot_general(
            v,
            w_ref[...],
            dimension_numbers=(((1,), (1,)), ((), ())),
            preferred_element_type=jnp.float32,
        )
        + b_ref[...]
    )


def _tc_linear(pooled, W, b2d):
    return pl.pallas_call(
        _linear_body,
        grid=(pl.cdiv(_VOCAB, _VBLK),),
        in_specs=[
            pl.BlockSpec((_BATCH, _HID), lambda j: (0, 0)),
            pl.BlockSpec((_VBLK, _HID), lambda j: (j, 0)),
            pl.BlockSpec((1, _VBLK), lambda j: (0, j)),
        ],
        out_specs=pl.BlockSpec((_BATCH, _VBLK), lambda j: (0, j)),
        out_shape=jax.ShapeDtypeStruct((_BATCH, _VOCAB), jnp.float32),
    )(pooled, W, b2d)


@jax.jit
def kernel(x, table, W, b):
    pooled = _sc_pooled_sum(x.reshape(_BATCH * _SEQ), table)
    return _tc_linear(pooled, W, b.reshape(1, _VOCAB))


# speedup vs baseline: 3.2854x; 2.2840x over previous
import functools

import jax
import jax.numpy as jnp
from jax import lax
from jax.experimental import pallas as pl
from jax.experimental.pallas import tpu as pltpu
from jax.experimental.pallas import tpu_sc as plsc

_VOCAB = 100000
_HID = 64
_BATCH = 1024
_SEQ = 200

_NC = 2
_NS = 16
_L = 16
_NW = _NC * _NS
_NBUF = 4
_NCOL = _HID // _L
_RPW = _BATCH // _NW

_sc_mesh = plsc.VectorSubcoreMesh(
    core_axis_name="c", subcore_axis_name="s", num_cores=_NC, num_subcores=_NS
)


@functools.partial(
    pl.kernel,
    out_type=jax.ShapeDtypeStruct((_BATCH, _HID), jnp.float32),
    mesh=_sc_mesh,
    scratch_types=[
        pltpu.VMEM((_RPW * _SEQ,), jnp.int32),
        pltpu.VMEM((_NBUF, _SEQ, _HID), jnp.float32),
        pltpu.VMEM((_RPW, _HID), jnp.float32),
        [pltpu.SemaphoreType.DMA] * _NBUF,
    ],
    compiler_params=pltpu.CompilerParams(use_tc_tiling_on_sc=False),
)
def _sc_pooled_sum(idx_hbm, table_hbm, out_hbm, idx_v, rows_v, pool_v, sems):
    wid = lax.axis_index("s") * _NC + lax.axis_index("c")
    row_base = wid * _RPW
    pltpu.sync_copy(idx_hbm.at[pl.ds(row_base * _SEQ, _RPW * _SEQ)], idx_v)

    def fire(g):
        buf = g % _NBUF
        return pltpu.async_copy(
            table_hbm.at[idx_v.at[pl.ds(g * _SEQ, _SEQ)]],
            rows_v.at[buf],
            sems[buf],
        )

    def reduce_into(g):
        buf = g % _NBUF

        def body(j8, acc):
            for k in range(8):
                j = j8 * 8 + k
                acc = tuple(
                    acc[c] + rows_v[buf, j, pl.ds(c * _L, _L)]
                    for c in range(_NCOL)
                )
            return acc

        z = jnp.zeros((_L,), jnp.float32)
        acc = lax.fori_loop(0, _SEQ // 8, body, (z,) * _NCOL)
        for c in range(_NCOL):
            pool_v[g, pl.ds(c * _L, _L)] = acc[c]

    cps = [None] * _NBUF
    for g0 in range(_NBUF - 1):
        cps[g0] = fire(g0)
    for g in range(_RPW):
        if g + _NBUF - 1 < _RPW:
            cps[(g + _NBUF - 1) % _NBUF] = fire(g + _NBUF - 1)
        cps[g % _NBUF].wait()
        reduce_into(g)

    pltpu.sync_copy(pool_v, out_hbm.at[pl.ds(row_base, _RPW)])


_VBLK = 5120
_NVB = pl.cdiv(_VOCAB, _VBLK)


def _linear_body(vec_ref, wt_ref, b_ref, out_ref):
    v = jnp.tanh(vec_ref[...] * (1.0 / _SEQ))
    out_ref[...] = (
        lax.dot_general(
            wt_ref[...],
            v,
            dimension_numbers=(((0,), (1,)), ((), ())),
            preferred_element_type=jnp.float32,
        )
        + b_ref[...]
    )


def _tc_linear_t(pooled, Wt, b_col):
    return pl.pallas_call(
        _linear_body,
        grid=(_NVB,),
        in_specs=[
            pl.BlockSpec((_BATCH, _HID), lambda j: (0, 0)),
            pl.BlockSpec((_HID, _VBLK), lambda j: (0, j)),
            pl.BlockSpec((_VBLK, 1), lambda j: (j, 0)),
        ],
        out_specs=pl.BlockSpec((_VBLK, _BATCH), lambda j: (j, 0)),
        out_shape=jax.ShapeDtypeStruct((_VOCAB, _BATCH), jnp.float32),
    )(pooled, Wt, b_col)


@jax.jit
def kernel(x, table, W, b):
    pooled = _sc_pooled_sum(x.reshape(_BATCH * _SEQ), table)
    pred_t = _tc_linear_t(pooled, W.T, b.reshape(_VOCAB, 1))
    return pred_t.T
